# baseline (device time: 131573 ns/iter reference)
import jax
import jax.numpy as jnp
from jax import lax
from jax.experimental import pallas as pl
from jax.experimental.pallas import tpu as pltpu

N_DEV = 32


def kernel(x, router_W, route_idx, expert_W):
    n, d = x.shape
    e_loc, _, h = expert_W.shape
    m = n // N_DEV

    def body(x_ref, rw_ref, idx_ref, ew_ref, out_ref,
             partial_ref, comm_ref, send_sems, recv_sems):
        my = lax.axis_index("i")
        left = lax.rem(my + N_DEV - 1, N_DEV)
        right = lax.rem(my + 1, N_DEV)

        barrier_sem = pltpu.get_barrier_semaphore()
        for nbr in (left, right):
            pl.semaphore_signal(
                barrier_sem, inc=1,
                device_id=(nbr,), device_id_type=pl.DeviceIdType.MESH,
            )
        pl.semaphore_wait(barrier_sem, 2)

        xf = x_ref[:, :]
        scores = jnp.dot(xf, rw_ref[:, :], preferred_element_type=jnp.float32)
        r0 = idx_ref[:, 0:1]
        r1 = idx_ref[:, 1:2]
        eids = lax.broadcasted_iota(jnp.int32, scores.shape, 1)
        s0 = jnp.sum(jnp.where(eids == r0, scores, 0.0), axis=1, keepdims=True)
        s1 = jnp.sum(jnp.where(eids == r1, scores, 0.0), axis=1, keepdims=True)
        w0 = 1.0 / (1.0 + jnp.exp(s1 - s0))
        w1 = 1.0 - w0

        for le in range(e_loc):
            eid = my * e_loc + le
            coef = jnp.where(r0 == eid, w0, 0.0) + jnp.where(r1 == eid, w1, 0.0)
            xg = (xf * coef).astype(jnp.bfloat16)
            wle = ew_ref[le, :, :].astype(jnp.bfloat16)
            acc = jnp.dot(xg, wle, preferred_element_type=jnp.float32)
            if le == 0:
                partial_ref[:, :] = acc
            else:
                partial_ref[:, :] = partial_ref[:, :] + acc

        c0 = lax.rem(my + N_DEV - 1, N_DEV)
        comm_ref[0, :, :] = partial_ref[pl.ds(c0 * m, m), :].astype(jnp.bfloat16)

        for s in range(N_DEV - 1):
            rdma = pltpu.make_async_remote_copy(
                src_ref=comm_ref.at[s],
                dst_ref=comm_ref.at[s + 1],
                send_sem=send_sems.at[s],
                recv_sem=recv_sems.at[s + 1],
                device_id=(right,),
                device_id_type=pl.DeviceIdType.MESH,
            )
            rdma.start()
            rdma.wait()
            if s < N_DEV - 2:
                c = lax.rem(my + 2 * N_DEV - 2 - s, N_DEV)
                comm_ref[s + 1, :, :] = (
                    comm_ref[s + 1, :, :].astype(jnp.float32)
                    + partial_ref[pl.ds(c * m, m), :]
                ).astype(jnp.bfloat16)

        out_ref[:, :] = (
            comm_ref[N_DEV - 1, :, :].astype(jnp.float32)
            + partial_ref[pl.ds(my * m, m), :]
        )

    return pl.pallas_call(
        body,
        out_shape=jax.ShapeDtypeStruct((m, h), jnp.float32),
        in_specs=[
            pl.BlockSpec(memory_space=pltpu.VMEM),
            pl.BlockSpec(memory_space=pltpu.VMEM),
            pl.BlockSpec(memory_space=pltpu.VMEM),
            pl.BlockSpec(memory_space=pltpu.VMEM),
        ],
        out_specs=pl.BlockSpec(memory_space=pltpu.VMEM),
        scratch_shapes=[
            pltpu.VMEM((n, h), jnp.float32),
            pltpu.VMEM((N_DEV, m, h), jnp.bfloat16),
            pltpu.SemaphoreType.DMA((N_DEV,)),
            pltpu.SemaphoreType.DMA((N_DEV,)),
        ],
        compiler_params=pltpu.CompilerParams(collective_id=0),
    )(x, router_W, route_idx, expert_W)


# device time: 119857 ns/iter; 1.0977x vs baseline; 1.0977x over previous
import jax
import jax.numpy as jnp
from jax import lax
from jax.experimental import pallas as pl
from jax.experimental.pallas import tpu as pltpu

N_DEV = 32


def kernel(x, router_W, route_idx, expert_W):
    n, d = x.shape
    e_loc, _, h = expert_W.shape
    m = n // N_DEV

    def body(x_ref, rw_ref, idx_ref, ew_ref, out_ref,
             coef_ref, comm_ref, send_sems, recv_sems):
        my = lax.axis_index("i")
        left = lax.rem(my + N_DEV - 1, N_DEV)
        right = lax.rem(my + 1, N_DEV)

        barrier_sem = pltpu.get_barrier_semaphore()
        for nbr in (left, right):
            pl.semaphore_signal(
                barrier_sem, inc=1,
                device_id=(nbr,), device_id_type=pl.DeviceIdType.MESH,
            )
        pl.semaphore_wait(barrier_sem, 2)

        scores = jnp.dot(x_ref[:, :], rw_ref[:, :],
                         preferred_element_type=jnp.float32)
        r0 = idx_ref[:, 0:1]
        r1 = idx_ref[:, 1:2]
        eids = lax.broadcasted_iota(jnp.int32, scores.shape, 1)
        s0 = jnp.sum(jnp.where(eids == r0, scores, 0.0), axis=1, keepdims=True)
        s1 = jnp.sum(jnp.where(eids == r1, scores, 0.0), axis=1, keepdims=True)
        w0 = 1.0 / (1.0 + jnp.exp(s1 - s0))
        w1 = 1.0 - w0
        for le in range(e_loc):
            eid = my * e_loc + le
            coef_ref[:, le:le + 1] = (
                jnp.where(r0 == eid, w0, 0.0) + jnp.where(r1 == eid, w1, 0.0)
            )

        wles = [ew_ref[le, :, :].astype(jnp.bfloat16) for le in range(e_loc)]

        def partial_chunk(c):
            rows = pl.ds(c * m, m)
            xc = x_ref[rows, :]
            acc = None
            for le in range(e_loc):
                xg = (xc * coef_ref[rows, le:le + 1]).astype(jnp.bfloat16)
                p = jnp.dot(xg, wles[le], preferred_element_type=jnp.float32)
                acc = p if acc is None else acc + p
            return acc

        c0 = lax.rem(my + N_DEV - 1, N_DEV)
        comm_ref[0, :, :] = partial_chunk(c0).astype(jnp.bfloat16)

        for s in range(N_DEV - 1):
            rdma = pltpu.make_async_remote_copy(
                src_ref=comm_ref.at[s],
                dst_ref=comm_ref.at[s + 1],
                send_sem=send_sems.at[s],
                recv_sem=recv_sems.at[s + 1],
                device_id=(right,),
                device_id_type=pl.DeviceIdType.MESH,
            )
            rdma.start()
            c = lax.rem(my + 2 * N_DEV - 2 - s, N_DEV)
            p = partial_chunk(c)
            rdma.wait()
            if s < N_DEV - 2:
                comm_ref[s + 1, :, :] = (
                    comm_ref[s + 1, :, :].astype(jnp.float32) + p
                ).astype(jnp.bfloat16)
            else:
                out_ref[:, :] = comm_ref[s + 1, :, :].astype(jnp.float32) + p

    return pl.pallas_call(
        body,
        out_shape=jax.ShapeDtypeStruct((m, h), jnp.float32),
        in_specs=[
            pl.BlockSpec(memory_space=pltpu.VMEM),
            pl.BlockSpec(memory_space=pltpu.VMEM),
            pl.BlockSpec(memory_space=pltpu.VMEM),
            pl.BlockSpec(memory_space=pltpu.VMEM),
        ],
        out_specs=pl.BlockSpec(memory_space=pltpu.VMEM),
        scratch_shapes=[
            pltpu.VMEM((n, e_loc), jnp.float32),
            pltpu.VMEM((N_DEV, m, h), jnp.bfloat16),
            pltpu.SemaphoreType.DMA((N_DEV,)),
            pltpu.SemaphoreType.DMA((N_DEV,)),
        ],
        compiler_params=pltpu.CompilerParams(collective_id=0),
    )(x, router_W, route_idx, expert_W)


# device time: 77825 ns/iter; 1.6906x vs baseline; 1.5401x over previous
import jax
import jax.numpy as jnp
from jax import lax
from jax.experimental import pallas as pl
from jax.experimental.pallas import tpu as pltpu

N_DEV = 32
NZ = 4
NP = 8
B = [0, 1, 2, 5, 6, 7, 4, 3]
INV_B = [0, 1, 2, 7, 6, 3, 4, 5]
NEXT_P = [1, 2, 5, 0, 3, 6, 7, 4]
PREV_P = [3, 0, 1, 4, 7, 2, 5, 6]


def _lut8(v, table):
    r = jnp.int32(table[0])
    for i in range(1, 8):
        r = jnp.where(v == i, jnp.int32(table[i]), r)
    return r


def kernel(x, router_W, route_idx, expert_W):
    n, d = x.shape
    e_loc, _, h = expert_W.shape
    m = n // N_DEV

    def body(x_ref, rw_ref, idx_ref, ew_ref, out_ref,
             coef_ref, comm1_ref, send2_ref, recv2_ref,
             send_sems1, recv_sems1, send_sems2, recv_sems2):
        my = lax.axis_index("i")
        z = my // NP
        p = lax.rem(my, NP)
        k = _lut8(p, INV_B)
        next_id = z * NP + _lut8(p, NEXT_P)
        prev_id = z * NP + _lut8(p, PREV_P)

        partners = [prev_id, next_id]
        for dz in range(1, NZ):
            partners.append(lax.rem(z + dz, NZ) * NP + p)
        barrier_sem = pltpu.get_barrier_semaphore()
        for nbr in partners:
            pl.semaphore_signal(
                barrier_sem, inc=1,
                device_id=(nbr,), device_id_type=pl.DeviceIdType.MESH,
            )
        pl.semaphore_wait(barrier_sem, len(partners))

        scores = jnp.dot(x_ref[:, :], rw_ref[:, :],
                         preferred_element_type=jnp.float32)
        r0 = idx_ref[:, 0:1]
        r1 = idx_ref[:, 1:2]
        eids = lax.broadcasted_iota(jnp.int32, scores.shape, 1)
        s0 = jnp.sum(jnp.where(eids == r0, scores, 0.0), axis=1, keepdims=True)
        s1 = jnp.sum(jnp.where(eids == r1, scores, 0.0), axis=1, keepdims=True)
        w0 = 1.0 / (1.0 + jnp.exp(s1 - s0))
        w1 = 1.0 - w0
        for le in range(e_loc):
            eid = my * e_loc + le
            coef_ref[:, le:le + 1] = (
                jnp.where(r0 == eid, w0, 0.0) + jnp.where(r1 == eid, w1, 0.0)
            )

        wles = [ew_ref[le, :, :].astype(jnp.bfloat16) for le in range(e_loc)]

        def partial_chunk(c):
            rows = pl.ds(c * m, m)
            xc = x_ref[rows, :]
            acc = None
            for le in range(e_loc):
                xg = (xc * coef_ref[rows, le:le + 1]).astype(jnp.bfloat16)
                pp = jnp.dot(xg, wles[le], preferred_element_type=jnp.float32)
                acc = pp if acc is None else acc + pp
            return acc

        def superblock_partials(j):
            q = _lut8(j, B)
            return [partial_chunk(zz * NP + q) for zz in range(NZ)]

        j0 = lax.rem(k + NP - 1, NP)
        for zz, pp in enumerate(superblock_partials(j0)):
            comm1_ref[0, zz, :, :] = pp.astype(jnp.bfloat16)

        finals = None
        for s in range(NP - 1):
            rdma = pltpu.make_async_remote_copy(
                src_ref=comm1_ref.at[s],
                dst_ref=comm1_ref.at[s + 1],
                send_sem=send_sems1.at[s],
                recv_sem=recv_sems1.at[s + 1],
                device_id=(next_id,),
                device_id_type=pl.DeviceIdType.MESH,
            )
            rdma.start()
            j = lax.rem(k + 2 * NP - 2 - s, NP)
            parts = superblock_partials(j)
            rdma.wait()
            if s < NP - 2:
                for zz in range(NZ):
                    comm1_ref[s + 1, zz, :, :] = (
                        comm1_ref[s + 1, zz, :, :].astype(jnp.float32)
                        + parts[zz]
                    ).astype(jnp.bfloat16)
            else:
                finals = [
                    comm1_ref[NP - 1, zz, :, :].astype(jnp.float32) + parts[zz]
                    for zz in range(NZ)
                ]

        sends = []
        for zt in range(NZ):
            @pl.when(z != zt)
            def _(zt=zt):
                send2_ref[zt, :, :] = finals[zt].astype(jnp.bfloat16)
                rdma2 = pltpu.make_async_remote_copy(
                    src_ref=send2_ref.at[zt],
                    dst_ref=recv2_ref.at[z],
                    send_sem=send_sems2.at[zt],
                    recv_sem=recv_sems2.at[z],
                    device_id=(zt * NP + p,),
                    device_id_type=pl.DeviceIdType.MESH,
                )
                rdma2.start()
                rdma2.wait_send()

        own = None
        for zz in range(NZ):
            t = jnp.where(z == zz, 1.0, 0.0) * finals[zz]
            own = t if own is None else own + t
        acc_out = own

        for zs in range(NZ):
            @pl.when(z != zs)
            def _(zs=zs):
                rdma_r = pltpu.make_async_remote_copy(
                    src_ref=send2_ref.at[zs],
                    dst_ref=recv2_ref.at[zs],
                    send_sem=send_sems2.at[zs],
                    recv_sem=recv_sems2.at[zs],
                    device_id=(my,),
                    device_id_type=pl.DeviceIdType.MESH,
                )
                rdma_r.wait_recv()
            acc_out = acc_out + jnp.where(
                z != zs, recv2_ref[zs, :, :].astype(jnp.float32), 0.0
            )

        out_ref[:, :] = acc_out

    return pl.pallas_call(
        body,
        out_shape=jax.ShapeDtypeStruct((m, h), jnp.float32),
        in_specs=[
            pl.BlockSpec(memory_space=pltpu.VMEM),
            pl.BlockSpec(memory_space=pltpu.VMEM),
            pl.BlockSpec(memory_space=pltpu.VMEM),
            pl.BlockSpec(memory_space=pltpu.VMEM),
        ],
        out_specs=pl.BlockSpec(memory_space=pltpu.VMEM),
        scratch_shapes=[
            pltpu.VMEM((n, e_loc), jnp.float32),
            pltpu.VMEM((NP, NZ, m, h), jnp.bfloat16),
            pltpu.VMEM((NZ, m, h), jnp.bfloat16),
            pltpu.VMEM((NZ, m, h), jnp.bfloat16),
            pltpu.SemaphoreType.DMA((NP,)),
            pltpu.SemaphoreType.DMA((NP,)),
            pltpu.SemaphoreType.DMA((NZ,)),
            pltpu.SemaphoreType.DMA((NZ,)),
        ],
        compiler_params=pltpu.CompilerParams(collective_id=0),
    )(x, router_W, route_idx, expert_W)


# device time: 59988 ns/iter; 2.1933x vs baseline; 1.2973x over previous
import jax
import jax.numpy as jnp
from jax import lax
from jax.experimental import pallas as pl
from jax.experimental.pallas import tpu as pltpu

N_DEV = 32
NZ = 4
NP = 8
B = [0, 1, 2, 5, 6, 7, 4, 3]
INV_B = [0, 1, 2, 7, 6, 3, 4, 5]
NEXT_P = [1, 2, 5, 0, 3, 6, 7, 4]
PREV_P = [3, 0, 1, 4, 7, 2, 5, 6]


def _lut8(v, table):
    r = jnp.int32(table[0])
    for i in range(1, 8):
        r = jnp.where(v == i, jnp.int32(table[i]), r)
    return r


def kernel(x, router_W, route_idx, expert_W):
    n, d = x.shape
    e_loc, _, h = expert_W.shape
    m = n // N_DEV

    h2 = h // 2

    def body(x_ref, rw_ref, idx_ref, ew_ref, out_ref,
             coef_ref, comm1r_ref, comm1l_ref, send2_ref, recv2_ref,
             send_sems1r, recv_sems1r, send_sems1l, recv_sems1l,
             send_sems2, recv_sems2):
        my = lax.axis_index("i")
        z = my // NP
        p = lax.rem(my, NP)
        k = _lut8(p, INV_B)
        next_id = z * NP + _lut8(p, NEXT_P)
        prev_id = z * NP + _lut8(p, PREV_P)

        partners = [prev_id, next_id]
        for dz in range(1, NZ):
            partners.append(lax.rem(z + dz, NZ) * NP + p)
        barrier_sem = pltpu.get_barrier_semaphore()
        for nbr in partners:
            pl.semaphore_signal(
                barrier_sem, inc=1,
                device_id=(nbr,), device_id_type=pl.DeviceIdType.MESH,
            )
        pl.semaphore_wait(barrier_sem, len(partners))

        scores = jnp.dot(x_ref[:, :], rw_ref[:, :],
                         preferred_element_type=jnp.float32)
        r0 = idx_ref[:, 0:1]
        r1 = idx_ref[:, 1:2]
        eids = lax.broadcasted_iota(jnp.int32, scores.shape, 1)
        s0 = jnp.sum(jnp.where(eids == r0, scores, 0.0), axis=1, keepdims=True)
        s1 = jnp.sum(jnp.where(eids == r1, scores, 0.0), axis=1, keepdims=True)
        w0 = 1.0 / (1.0 + jnp.exp(s1 - s0))
        w1 = 1.0 - w0
        for le in range(e_loc):
            eid = my * e_loc + le
            coef_ref[:, le:le + 1] = (
                jnp.where(r0 == eid, w0, 0.0) + jnp.where(r1 == eid, w1, 0.0)
            )

        wles = [ew_ref[le, :, :].astype(jnp.bfloat16) for le in range(e_loc)]

        def partial_chunk(c, lo, hi):
            rows = pl.ds(c * m, m)
            xc = x_ref[rows, :]
            acc = None
            for le in range(e_loc):
                xg = (xc * coef_ref[rows, le:le + 1]).astype(jnp.bfloat16)
                pp = jnp.dot(xg, wles[le][:, lo:hi],
                             preferred_element_type=jnp.float32)
                acc = pp if acc is None else acc + pp
            return acc

        def superblock_partials(j, lo, hi):
            q = _lut8(j, B)
            return [partial_chunk(zz * NP + q, lo, hi) for zz in range(NZ)]

        jr0 = lax.rem(k + NP - 1, NP)
        for zz, pp in enumerate(superblock_partials(jr0, 0, h2)):
            comm1r_ref[0, zz, :, :] = pp.astype(jnp.bfloat16)
        jl0 = lax.rem(k + 1, NP)
        for zz, pp in enumerate(superblock_partials(jl0, h2, h)):
            comm1l_ref[0, zz, :, :] = pp.astype(jnp.bfloat16)

        finals_r = finals_l = None
        for s in range(NP - 1):
            rdma_r = pltpu.make_async_remote_copy(
                src_ref=comm1r_ref.at[s],
                dst_ref=comm1r_ref.at[s + 1],
                send_sem=send_sems1r.at[s],
                recv_sem=recv_sems1r.at[s + 1],
                device_id=(next_id,),
                device_id_type=pl.DeviceIdType.MESH,
            )
            rdma_r.start()
            rdma_l = pltpu.make_async_remote_copy(
                src_ref=comm1l_ref.at[s],
                dst_ref=comm1l_ref.at[s + 1],
                send_sem=send_sems1l.at[s],
                recv_sem=recv_sems1l.at[s + 1],
                device_id=(prev_id,),
                device_id_type=pl.DeviceIdType.MESH,
            )
            rdma_l.start()
            jr = lax.rem(k + 2 * NP - 2 - s, NP)
            parts_r = superblock_partials(jr, 0, h2)
            jl = lax.rem(k + 2 + s, NP)
            parts_l = superblock_partials(jl, h2, h)
            rdma_r.wait()
            rdma_l.wait()
            if s < NP - 2:
                for zz in range(NZ):
                    comm1r_ref[s + 1, zz, :, :] = (
                        comm1r_ref[s + 1, zz, :, :].astype(jnp.float32)
                        + parts_r[zz]
                    ).astype(jnp.bfloat16)
                    comm1l_ref[s + 1, zz, :, :] = (
                        comm1l_ref[s + 1, zz, :, :].astype(jnp.float32)
                        + parts_l[zz]
                    ).astype(jnp.bfloat16)
            else:
                finals_r = [
                    comm1r_ref[NP - 1, zz, :, :].astype(jnp.float32)
                    + parts_r[zz]
                    for zz in range(NZ)
                ]
                finals_l = [
                    comm1l_ref[NP - 1, zz, :, :].astype(jnp.float32)
                    + parts_l[zz]
                    for zz in range(NZ)
                ]

        for zt in range(NZ):
            @pl.when(z != zt)
            def _(zt=zt):
                send2_ref[zt, :, 0:h2] = finals_r[zt].astype(jnp.bfloat16)
                send2_ref[zt, :, h2:h] = finals_l[zt].astype(jnp.bfloat16)
                rdma2 = pltpu.make_async_remote_copy(
                    src_ref=send2_ref.at[zt],
                    dst_ref=recv2_ref.at[z],
                    send_sem=send_sems2.at[zt],
                    recv_sem=recv_sems2.at[z],
                    device_id=(zt * NP + p,),
                    device_id_type=pl.DeviceIdType.MESH,
                )
                rdma2.start()
                rdma2.wait_send()

        acc_r = acc_l = None
        for zz in range(NZ):
            sel = jnp.where(z == zz, 1.0, 0.0)
            tr = sel * finals_r[zz]
            tl = sel * finals_l[zz]
            acc_r = tr if acc_r is None else acc_r + tr
            acc_l = tl if acc_l is None else acc_l + tl

        for zs in range(NZ):
            @pl.when(z != zs)
            def _(zs=zs):
                rdma_w = pltpu.make_async_remote_copy(
                    src_ref=send2_ref.at[zs],
                    dst_ref=recv2_ref.at[zs],
                    send_sem=send_sems2.at[zs],
                    recv_sem=recv_sems2.at[zs],
                    device_id=(my,),
                    device_id_type=pl.DeviceIdType.MESH,
                )
                rdma_w.wait_recv()
            mask = jnp.where(z != zs, 1.0, 0.0)
            acc_r = acc_r + mask * recv2_ref[zs, :, 0:h2].astype(jnp.float32)
            acc_l = acc_l + mask * recv2_ref[zs, :, h2:h].astype(jnp.float32)

        out_ref[:, 0:h2] = acc_r
        out_ref[:, h2:h] = acc_l

    return pl.pallas_call(
        body,
        out_shape=jax.ShapeDtypeStruct((m, h), jnp.float32),
        in_specs=[
            pl.BlockSpec(memory_space=pltpu.VMEM),
            pl.BlockSpec(memory_space=pltpu.VMEM),
            pl.BlockSpec(memory_space=pltpu.VMEM),
            pl.BlockSpec(memory_space=pltpu.VMEM),
        ],
        out_specs=pl.BlockSpec(memory_space=pltpu.VMEM),
        scratch_shapes=[
            pltpu.VMEM((n, e_loc), jnp.float32),
            pltpu.VMEM((NP, NZ, m, h // 2), jnp.bfloat16),
            pltpu.VMEM((NP, NZ, m, h // 2), jnp.bfloat16),
            pltpu.VMEM((NZ, m, h), jnp.bfloat16),
            pltpu.VMEM((NZ, m, h), jnp.bfloat16),
            pltpu.SemaphoreType.DMA((NP,)),
            pltpu.SemaphoreType.DMA((NP,)),
            pltpu.SemaphoreType.DMA((NP,)),
            pltpu.SemaphoreType.DMA((NP,)),
            pltpu.SemaphoreType.DMA((NZ,)),
            pltpu.SemaphoreType.DMA((NZ,)),
        ],
        compiler_params=pltpu.CompilerParams(collective_id=0),
    )(x, router_W, route_idx, expert_W)


# device time: 58545 ns/iter; 2.2474x vs baseline; 1.0246x over previous
import jax
import jax.numpy as jnp
from jax import lax
from jax.experimental import pallas as pl
from jax.experimental.pallas import tpu as pltpu

N_DEV = 32
NZ = 4
NP = 8
B = [0, 1, 2, 5, 6, 7, 4, 3]
INV_B = [0, 1, 2, 7, 6, 3, 4, 5]
NEXT_P = [1, 2, 5, 0, 3, 6, 7, 4]
PREV_P = [3, 0, 1, 4, 7, 2, 5, 6]


def _lut8(v, table):
    r = jnp.int32(table[0])
    for i in range(1, 8):
        r = jnp.where(v == i, jnp.int32(table[i]), r)
    return r


def kernel(x, router_W, route_idx, expert_W):
    n, d = x.shape
    e_loc, _, h = expert_W.shape
    m = n // N_DEV

    h2 = h // 2

    def body(x_ref, rw_ref, idx_ref, ew_ref, out_ref,
             coef_ref, xg_ref, cg_ref, comm1r_ref, comm1l_ref,
             send2_ref, recv2_ref,
             send_sems1r, recv_sems1r, send_sems1l, recv_sems1l,
             send_sems2, recv_sems2):
        my = lax.axis_index("i")
        z = my // NP
        p = lax.rem(my, NP)
        k = _lut8(p, INV_B)
        next_id = z * NP + _lut8(p, NEXT_P)
        prev_id = z * NP + _lut8(p, PREV_P)

        recv2_ref[...] = jnp.zeros((NZ, m, h), jnp.bfloat16)

        partners = [prev_id, next_id]
        for dz in range(1, NZ):
            partners.append(lax.rem(z + dz, NZ) * NP + p)
        barrier_sem = pltpu.get_barrier_semaphore()
        for nbr in partners:
            pl.semaphore_signal(
                barrier_sem, inc=1,
                device_id=(nbr,), device_id_type=pl.DeviceIdType.MESH,
            )
        pl.semaphore_wait(barrier_sem, len(partners))

        scores = jnp.dot(x_ref[:, :], rw_ref[:, :],
                         preferred_element_type=jnp.float32)
        r0 = idx_ref[:, 0:1]
        r1 = idx_ref[:, 1:2]
        eids = lax.broadcasted_iota(jnp.int32, scores.shape, 1)
        s0 = jnp.sum(jnp.where(eids == r0, scores, 0.0), axis=1, keepdims=True)
        s1 = jnp.sum(jnp.where(eids == r1, scores, 0.0), axis=1, keepdims=True)
        w0 = 1.0 / (1.0 + jnp.exp(s1 - s0))
        w1 = 1.0 - w0
        for le in range(e_loc):
            eid = my * e_loc + le
            coef_ref[:, le:le + 1] = (
                jnp.where(r0 == eid, w0, 0.0) + jnp.where(r1 == eid, w1, 0.0)
            )

        wles = [ew_ref[le, :, :].astype(jnp.bfloat16) for le in range(e_loc)]

        def superblock_partials(j, lo, hi):
            q = _lut8(j, B)
            for zz in range(NZ):
                rows = pl.ds((zz * NP + q) * m, m)
                xg_ref[zz * m:(zz + 1) * m, :] = (
                    x_ref[rows, :].astype(jnp.bfloat16))
                cg_ref[zz * m:(zz + 1) * m, :] = coef_ref[rows, :]
            acc = None
            for le in range(e_loc):
                mm = jnp.dot(xg_ref[:, :], wles[le][:, lo:hi],
                             preferred_element_type=jnp.float32)
                pp = cg_ref[:, le:le + 1] * mm
                acc = pp if acc is None else acc + pp
            return [acc[zz * m:(zz + 1) * m, :] for zz in range(NZ)]

        jr0 = lax.rem(k + NP - 1, NP)
        for zz, pp in enumerate(superblock_partials(jr0, 0, h2)):
            comm1r_ref[0, zz, :, :] = pp.astype(jnp.bfloat16)
        jl0 = lax.rem(k + 1, NP)
        for zz, pp in enumerate(superblock_partials(jl0, h2, h)):
            comm1l_ref[0, zz, :, :] = pp.astype(jnp.bfloat16)

        finals_r = finals_l = None
        for s in range(NP - 1):
            rdma_r = pltpu.make_async_remote_copy(
                src_ref=comm1r_ref.at[s],
                dst_ref=comm1r_ref.at[s + 1],
                send_sem=send_sems1r.at[s],
                recv_sem=recv_sems1r.at[s + 1],
                device_id=(next_id,),
                device_id_type=pl.DeviceIdType.MESH,
            )
            rdma_r.start()
            rdma_l = pltpu.make_async_remote_copy(
                src_ref=comm1l_ref.at[s],
                dst_ref=comm1l_ref.at[s + 1],
                send_sem=send_sems1l.at[s],
                recv_sem=recv_sems1l.at[s + 1],
                device_id=(prev_id,),
                device_id_type=pl.DeviceIdType.MESH,
            )
            rdma_l.start()
            jr = lax.rem(k + 2 * NP - 2 - s, NP)
            parts_r = superblock_partials(jr, 0, h2)
            jl = lax.rem(k + 2 + s, NP)
            parts_l = superblock_partials(jl, h2, h)
            rdma_r.wait()
            rdma_l.wait()
            if s < NP - 2:
                for zz in range(NZ):
                    comm1r_ref[s + 1, zz, :, :] = (
                        comm1r_ref[s + 1, zz, :, :].astype(jnp.float32)
                        + parts_r[zz]
                    ).astype(jnp.bfloat16)
                    comm1l_ref[s + 1, zz, :, :] = (
                        comm1l_ref[s + 1, zz, :, :].astype(jnp.float32)
                        + parts_l[zz]
                    ).astype(jnp.bfloat16)
            else:
                finals_r = [
                    comm1r_ref[NP - 1, zz, :, :].astype(jnp.float32)
                    + parts_r[zz]
                    for zz in range(NZ)
                ]
                finals_l = [
                    comm1l_ref[NP - 1, zz, :, :].astype(jnp.float32)
                    + parts_l[zz]
                    for zz in range(NZ)
                ]

        for zt in range(NZ):
            @pl.when(z != zt)
            def _(zt=zt):
                send2_ref[zt, :, 0:h2] = finals_r[zt].astype(jnp.bfloat16)
                send2_ref[zt, :, h2:h] = finals_l[zt].astype(jnp.bfloat16)
                rdma2 = pltpu.make_async_remote_copy(
                    src_ref=send2_ref.at[zt],
                    dst_ref=recv2_ref.at[z],
                    send_sem=send_sems2.at[zt],
                    recv_sem=recv_sems2.at[z],
                    device_id=(zt * NP + p,),
                    device_id_type=pl.DeviceIdType.MESH,
                )
                rdma2.start()
                rdma2.wait_send()

        acc_r = acc_l = None
        for zz in range(NZ):
            sel = jnp.where(z == zz, 1.0, 0.0)
            tr = sel * finals_r[zz]
            tl = sel * finals_l[zz]
            acc_r = tr if acc_r is None else acc_r + tr
            acc_l = tl if acc_l is None else acc_l + tl

        for zs in range(NZ):
            @pl.when(z != zs)
            def _(zs=zs):
                rdma_w = pltpu.make_async_remote_copy(
                    src_ref=send2_ref.at[zs],
                    dst_ref=recv2_ref.at[zs],
                    send_sem=send_sems2.at[zs],
                    recv_sem=recv_sems2.at[zs],
                    device_id=(my,),
                    device_id_type=pl.DeviceIdType.MESH,
                )
                rdma_w.wait_recv()
            mask = jnp.where(z != zs, 1.0, 0.0)
            acc_r = acc_r + mask * recv2_ref[zs, :, 0:h2].astype(jnp.float32)
            acc_l = acc_l + mask * recv2_ref[zs, :, h2:h].astype(jnp.float32)

        out_ref[:, 0:h2] = acc_r
        out_ref[:, h2:h] = acc_l

    return pl.pallas_call(
        body,
        out_shape=jax.ShapeDtypeStruct((m, h), jnp.float32),
        in_specs=[
            pl.BlockSpec(memory_space=pltpu.VMEM),
            pl.BlockSpec(memory_space=pltpu.VMEM),
            pl.BlockSpec(memory_space=pltpu.VMEM),
            pl.BlockSpec(memory_space=pltpu.VMEM),
        ],
        out_specs=pl.BlockSpec(memory_space=pltpu.VMEM),
        scratch_shapes=[
            pltpu.VMEM((n, e_loc), jnp.float32),
            pltpu.VMEM((NZ * m, d), jnp.bfloat16),
            pltpu.VMEM((NZ * m, e_loc), jnp.float32),
            pltpu.VMEM((NP, NZ, m, h // 2), jnp.bfloat16),
            pltpu.VMEM((NP, NZ, m, h // 2), jnp.bfloat16),
            pltpu.VMEM((NZ, m, h), jnp.bfloat16),
            pltpu.VMEM((NZ, m, h), jnp.bfloat16),
            pltpu.SemaphoreType.DMA((NP,)),
            pltpu.SemaphoreType.DMA((NP,)),
            pltpu.SemaphoreType.DMA((NP,)),
            pltpu.SemaphoreType.DMA((NP,)),
            pltpu.SemaphoreType.DMA((NZ,)),
            pltpu.SemaphoreType.DMA((NZ,)),
        ],
        compiler_params=pltpu.CompilerParams(collective_id=0),
    )(x, router_W, route_idx, expert_W)
